# baseline (speedup 1.0000x reference)
import functools

import jax
import jax.numpy as jnp
from jax.experimental import pallas as pl
from jax.experimental.pallas import tpu as pltpu

_BM = 1024


def _linear_kernel(w0_ref, w1_ref, x_ref, b_ref, o_ref):
    half = w0_ref.shape[0]
    xv = x_ref[...]
    dn = (((1,), (1,)), ((), ()))
    o_ref[:, :half] = jax.lax.dot_general(
        xv, w0_ref[...], dimension_numbers=dn,
        preferred_element_type=jnp.float32) + b_ref[:, :half]
    o_ref[:, half:] = jax.lax.dot_general(
        xv, w1_ref[...], dimension_numbers=dn,
        preferred_element_type=jnp.float32) + b_ref[:, half:]


@functools.partial(jax.jit, static_argnames=("bm",))
def _forward(x, w, b, *, bm):
    M, K = x.shape
    N = w.shape[0]
    b_row = b.reshape(1, N)
    steps = M // bm // 2
    grid = (2, steps)
    out = pl.pallas_call(
        _linear_kernel,
        out_shape=jax.ShapeDtypeStruct((M, N), jnp.float32),
        grid=grid,
        in_specs=[
            pl.BlockSpec((N // 2, K), lambda i, j: (0, 0)),
            pl.BlockSpec((N // 2, K), lambda i, j: (1, 0)),
            pl.BlockSpec((bm, K), lambda i, j: (i * steps + j, 0)),
            pl.BlockSpec((1, N), lambda i, j: (0, 0)),
        ],
        out_specs=pl.BlockSpec((bm, N), lambda i, j: (i * steps + j, 0)),
        compiler_params=pltpu.CompilerParams(
            dimension_semantics=("parallel", "arbitrary"),
            vmem_limit_bytes=62 * 1024 * 1024),
        cost_estimate=pl.CostEstimate(
            flops=2 * M * N * K,
            bytes_accessed=4 * M * K + 4 * K * N + 4 * M * N,
            transcendentals=0),
    )(w, w, x, b_row)
    return out


def kernel(x, w, b):
    bm = _BM if x.shape[0] % (2 * _BM) == 0 else 8
    return _forward(x, w, b, bm=bm)


# speedup vs baseline: 1.0004x; 1.0004x over previous
import functools

import jax
import jax.numpy as jnp
from jax.experimental import pallas as pl
from jax.experimental.pallas import tpu as pltpu

_BM = 1024


def _linear_kernel(w_ref, x_ref, b_ref, o_ref):
    acc = jax.lax.dot_general(
        x_ref[...], w_ref[...],
        dimension_numbers=(((1,), (1,)), ((), ())),
        preferred_element_type=jnp.float32)
    o_ref[...] = acc + b_ref[...]


@functools.partial(jax.jit, static_argnames=("bm",))
def _forward(x, w, b, *, bm):
    M, K = x.shape
    N = w.shape[0]
    b_row = b.reshape(1, N)
    steps = M // bm // 2
    grid = (2, steps)
    out = pl.pallas_call(
        _linear_kernel,
        out_shape=jax.ShapeDtypeStruct((M, N), jnp.float32),
        grid=grid,
        in_specs=[
            pl.BlockSpec((N, K), lambda i, j: (0, 0)),
            pl.BlockSpec((bm, K), lambda i, j: (i * steps + j, 0)),
            pl.BlockSpec((1, N), lambda i, j: (0, 0)),
        ],
        out_specs=pl.BlockSpec((bm, N), lambda i, j: (i * steps + j, 0)),
        compiler_params=pltpu.CompilerParams(
            dimension_semantics=("parallel", "arbitrary"),
            vmem_limit_bytes=62 * 1024 * 1024),
        cost_estimate=pl.CostEstimate(
            flops=2 * M * N * K,
            bytes_accessed=4 * M * K + 4 * K * N + 4 * M * N,
            transcendentals=0),
    )(w, x, b_row)
    return out


def kernel(x, w, b):
    bm = _BM if x.shape[0] % (2 * _BM) == 0 else 8
    return _forward(x, w, b, bm=bm)
